# Initial kernel scaffold; baseline (speedup 1.0000x reference)
import jax
import jax.numpy as jnp
from jax.experimental import pallas as pl


def kernel(x_in, edge_index, edge_attr, params):
    raise NotImplementedError("write your pallas kernel here")


# speedup vs baseline: 2.9870x; 2.9870x over previous
import functools

import jax
import jax.numpy as jnp
from jax import lax
from jax.experimental import pallas as pl
from jax.experimental.pallas import tpu as pltpu
from jax.experimental.pallas import tpu_sc as plsc

NN = 10000
NE = 320000
HID = 32
OUT = 128
HEADS = 4
WEX = 144

NC, NS = 2, 16
NW = NC * NS
EPW = NE // NW
K = 80
NCHUNK = EPW // K
NB = 5
KS = 40
NCHUNKS = EPW // KS
RPT = NN // NS
ZR = 25

BN = 2000
BEB = 2000


def _row(bs, d):
    return pl.BlockSpec((bs, d), lambda i: (i, 0))


def _full(shape):
    nd = len(shape)
    return pl.BlockSpec(shape, lambda i: (0,) * nd)


def _ln(x, w, b, eps=1e-5):
    m = jnp.mean(x, -1, keepdims=True)
    v = jnp.mean((x - m) * (x - m), -1, keepdims=True)
    return (x - m) * jax.lax.rsqrt(v + eps) * w + b


def _mm(x, w):
    return jnp.dot(x, w, preferred_element_type=jnp.float32)


def _sc_gather(ts, td, src, dst, ds_, dd_):
    mesh = plsc.VectorSubcoreMesh(core_axis_name="c", subcore_axis_name="s")
    scratch = [pltpu.VMEM((EPW,), jnp.int32), pltpu.VMEM((EPW,), jnp.int32)]
    scratch += [pltpu.VMEM((K, ds_), jnp.float32) for _ in range(NB)]
    scratch += [pltpu.VMEM((K, dd_), jnp.float32) for _ in range(NB)]
    scratch += [pltpu.SemaphoreType.DMA for _ in range(4 * NB)]

    @functools.partial(
        pl.kernel, mesh=mesh,
        out_type=[jax.ShapeDtypeStruct((NE, ds_), jnp.float32),
                  jax.ShapeDtypeStruct((NE, dd_), jnp.float32)],
        compiler_params=pltpu.CompilerParams(use_tc_tiling_on_sc=False),
        scratch_types=scratch)
    def k(ts_hbm, td_hbm, src_hbm, dst_hbm, gs_out, gd_out, *sc):
        idx_s, idx_d = sc[0], sc[1]
        sbuf = sc[2:2 + NB]
        dbuf = sc[2 + NB:2 + 2 * NB]
        gsem_s = sc[2 + 2 * NB:2 + 3 * NB]
        gsem_d = sc[2 + 3 * NB:2 + 4 * NB]
        ssem_s = sc[2 + 4 * NB:2 + 5 * NB]
        ssem_d = sc[2 + 5 * NB:2 + 6 * NB]
        wid = lax.axis_index("s") * NC + lax.axis_index("c")
        base = wid * EPW
        pltpu.sync_copy(src_hbm.at[pl.ds(base, EPW)], idx_s)
        pltpu.sync_copy(dst_hbm.at[pl.ds(base, EPW)], idx_d)

        def rnd(r, _):
            off = r * (NB * K)
            for b in range(NB):
                co = off + b * K
                pltpu.async_copy(ts_hbm.at[idx_s.at[pl.ds(co, K)]], sbuf[b], gsem_s[b])
                pltpu.async_copy(td_hbm.at[idx_d.at[pl.ds(co, K)]], dbuf[b], gsem_d[b])
            for b in range(NB):
                co = off + b * K
                pltpu.make_async_copy(ts_hbm.at[idx_s.at[pl.ds(co, K)]], sbuf[b], gsem_s[b]).wait()
                pltpu.make_async_copy(td_hbm.at[idx_d.at[pl.ds(co, K)]], dbuf[b], gsem_d[b]).wait()
            for b in range(NB):
                co = off + b * K
                pltpu.async_copy(sbuf[b], gs_out.at[pl.ds(base + co, K)], ssem_s[b])
                pltpu.async_copy(dbuf[b], gd_out.at[pl.ds(base + co, K)], ssem_d[b])
            for b in range(NB):
                co = off + b * K
                pltpu.make_async_copy(sbuf[b], gs_out.at[pl.ds(base + co, K)], ssem_s[b]).wait()
                pltpu.make_async_copy(dbuf[b], gd_out.at[pl.ds(base + co, K)], ssem_d[b]).wait()
            return 0

        lax.fori_loop(0, NCHUNK // NB, rnd, 0)

    return k(ts, td, src, dst)


def _sc_scatter(wex, dst3):
    mesh = plsc.VectorSubcoreMesh(core_axis_name="c", subcore_axis_name="s")
    scratch = [pltpu.VMEM((NB, KS), jnp.int32)]
    scratch += [pltpu.VMEM((KS, WEX), jnp.float32) for _ in range(NB)]
    scratch += [pltpu.VMEM((ZR, WEX), jnp.float32)]
    scratch += [pltpu.VMEM_SHARED((NN, WEX), jnp.float32)]
    scratch += [pltpu.SemaphoreType.DMA for _ in range(2 * NB)]

    @functools.partial(
        pl.kernel, mesh=mesh,
        out_type=jax.ShapeDtypeStruct((NC, NN, WEX), jnp.float32),
        compiler_params=pltpu.CompilerParams(use_tc_tiling_on_sc=False),
        scratch_types=scratch)
    def k(wex_hbm, dst_hbm, out_hbm, idxb, *sc):
        buf = sc[:NB]
        zbuf = sc[NB]
        acc = sc[NB + 1]
        lsem = sc[NB + 2:NB + 2 + NB]
        ssem = sc[NB + 2 + NB:NB + 2 + 2 * NB]
        cid = lax.axis_index("c")
        sid = lax.axis_index("s")
        wid = sid * NC + cid

        def zrow(i, _):
            for j in range(WEX // 16):
                zbuf[i, pl.ds(j * 16, 16)] = jnp.zeros((16,), jnp.float32)
            return 0

        lax.fori_loop(0, ZR, zrow, 0)
        for c in range(RPT // ZR):
            pltpu.sync_copy(zbuf, acc.at[pl.ds(sid * RPT + c * ZR, ZR)])
        plsc.subcore_barrier()

        base = wid * EPW

        def rnd(r, _):
            pltpu.sync_copy(dst_hbm.at[wid, pl.ds(r * NB, NB)], idxb)
            for b in range(NB):
                j = r * NB + b
                pltpu.async_copy(wex_hbm.at[pl.ds(base + j * KS, KS)], buf[b], lsem[b])
            for b in range(NB):
                j = r * NB + b
                pltpu.make_async_copy(wex_hbm.at[pl.ds(base + j * KS, KS)], buf[b], lsem[b]).wait()
            for b in range(NB):
                pltpu.async_copy(buf[b], acc.at[idxb.at[b]], ssem[b], add=True)
            for b in range(NB):
                pltpu.make_async_copy(buf[b], acc.at[idxb.at[b]], ssem[b]).wait()
            return 0

        lax.fori_loop(0, NCHUNKS // NB, rnd, 0)
        plsc.subcore_barrier()
        pltpu.sync_copy(acc.at[pl.ds(sid * RPT, RPT)],
                        out_hbm.at[cid, pl.ds(sid * RPT, RPT)])

    return k(wex, dst3)


def _node_encode(x_in, dummy, ne1w, ne1b, ne2w, ne2b, ws0, wd0):
    def body(x_ref, dm, w1, b1, w2, b2, wsr, wdr, x0_o, ts_o, td_o):
        x = x_ref[...]
        inv = x[:, 0:1] == -999.0
        xc = jnp.where(inv, dm[...], x)
        h = jnp.maximum(_mm(xc, w1[...]) + b1[...], 0.0)
        x0 = _mm(h, w2[...]) + b2[...]
        x0_o[...] = x0
        ts_o[...] = _mm(x0, wsr[...])
        td_o[...] = _mm(x0, wdr[...])

    grid = (NN // BN,)
    return pl.pallas_call(
        body, grid=grid,
        in_specs=[_row(BN, 128), _full((1, 128)), _full((128, 32)), _full((1, 32)),
                  _full((32, 32)), _full((1, 32)), _full((32, 160)), _full((32, 32))],
        out_specs=[_row(BN, 32), _row(BN, 160), _row(BN, 32)],
        out_shape=[jax.ShapeDtypeStruct((NN, 32), jnp.float32),
                   jax.ShapeDtypeStruct((NN, 160), jnp.float32),
                   jax.ShapeDtypeStruct((NN, 32), jnp.float32)],
    )(x_in, dummy, ne1w, ne1b, ne2w, ne2b, ws0, wd0)


def _edge_encode(ea, w1, b1, w2, b2):
    def body(e_ref, w1r, b1r, w2r, b2r, o):
        h = jnp.maximum(_mm(e_ref[...], w1r[...]) + b1r[...], 0.0)
        o[...] = _mm(h, w2r[...]) + b2r[...]

    return pl.pallas_call(
        body, grid=(NE // BEB,),
        in_specs=[_row(BEB, 16), _full((16, 32)), _full((1, 32)),
                  _full((32, 32)), _full((1, 32))],
        out_specs=_row(BEB, 32),
        out_shape=jax.ShapeDtypeStruct((NE, 32), jnp.float32),
    )(ea, w1, b1, w2, b2)


def _edge_stage(gs, gd, e_prev, W, has_update):
    ds_ = 192 if has_update else 160
    dd_ = 64 if has_update else 32

    def body(*refs):
        if has_update:
            (gs_r, gd_r, ep_r, ue, bu1, uw2, bu2, lnw, lnb,
             ae, ba1, a2p, ba2, me, bm1, m2, bm2, b8, wex_o, e_o) = refs
        else:
            (gs_r, gd_r, ep_r,
             ae, ba1, a2p, ba2, me, bm1, m2, bm2, b8, wex_o) = refs
        gs_v = gs_r[...]
        gd_v = gd_r[...]
        ep = ep_r[...]
        if has_update:
            uh = jnp.maximum(gd_v[:, 32:64] + gs_v[:, 160:192]
                             + _mm(ep, ue[...]) + bu1[...], 0.0)
            e = _ln(ep + _mm(uh, uw2[...]) + bu2[...], lnw[...], lnb[...])
            e_o[...] = e
        else:
            e = ep
        ah = jnp.maximum(gd_v[:, 0:32] + gs_v[:, 128:160]
                         + _mm(e, ae[...]) + ba1[...], 0.0)
        ex8 = jnp.exp(_mm(ah, a2p[...]) + ba2[...])
        mh = jnp.maximum(gs_v[:, 0:128] + _mm(e, me[...]) + bm1[...], 0.0)
        m = _mm(mh, m2[...]) + bm2[...]
        wex_o[:, 0:128] = m * _mm(ex8, b8[...])
        wex_o[:, 128:136] = ex8
        wex_o[:, 136:144] = jnp.zeros((BEB, 8), jnp.float32)

    in_specs = [_row(BEB, ds_), _row(BEB, dd_), _row(BEB, 32)]
    args = [gs, gd, e_prev]
    if has_update:
        in_specs += [_full((32, 32)), _full((1, 32)), _full((32, 32)), _full((1, 32)),
                     _full((1, 32)), _full((1, 32))]
        args += [W['ue'], W['bu1'], W['uw2'], W['bu2'], W['lnw'], W['lnb']]
    in_specs += [_full((32, 32)), _full((1, 32)), _full((32, 8)), _full((1, 8)),
                 _full((32, 128)), _full((1, 128)), _full((128, 128)), _full((1, 128)),
                 _full((8, 128))]
    args += [W['ae'], W['ba1'], W['a2p'], W['ba2'], W['me'], W['bm1'],
             W['m2'], W['bm2'], W['b8']]
    out_specs = [_row(BEB, WEX)]
    out_shape = [jax.ShapeDtypeStruct((NE, WEX), jnp.float32)]
    if has_update:
        out_specs += [_row(BEB, 32)]
        out_shape += [jax.ShapeDtypeStruct((NE, 32), jnp.float32)]
    res = pl.pallas_call(body, grid=(NE // BEB,), in_specs=in_specs,
                         out_specs=out_specs, out_shape=out_shape)(*args)
    return res if has_update else (res[0], None)


def _combine(parts, skip_x, skip_w, skip_b, lnw, lnb, b8, wsn, wdn,
             x1=None, slw=None, slb=None):
    ds_n = wsn.shape[1]
    dd_n = wdn.shape[1]
    has_sw = skip_w is not None
    has_l = x1 is not None
    skip_d = skip_x.shape[1]

    def body(*refs):
        i = 0
        p_r = refs[i]; i += 1
        sk_r = refs[i]; i += 1
        if has_sw:
            skw = refs[i]; i += 1
            skb = refs[i]; i += 1
        if has_l:
            x1_r = refs[i]; i += 1
            slw_r = refs[i]; i += 1
            slb_r = refs[i]; i += 1
        lnw_r = refs[i]; i += 1
        lnb_r = refs[i]; i += 1
        b8_r = refs[i]; i += 1
        wsn_r = refs[i]; i += 1
        wdn_r = refs[i]; i += 1
        x_o, ts_o, td_o = refs[i], refs[i + 1], refs[i + 2]

        S = p_r[0] + p_r[1]
        den = _mm(S[:, 128:136], b8_r[...]) + 1e-16
        agg = S[:, 0:128] / den
        if has_sw:
            sk = _mm(sk_r[...], skw[...]) + skb[...]
        else:
            sk = sk_r[...]
        x = jnp.maximum(_ln(agg + sk, lnw_r[...], lnb_r[...]), 0.0)
        if has_l:
            x = x + _mm(x1_r[...], slw_r[...]) + slb_r[...]
        x_o[...] = x
        ts_o[...] = _mm(x, wsn_r[...])
        td_o[...] = _mm(x, wdn_r[...])

    in_specs = [pl.BlockSpec((2, BN, WEX), lambda i: (0, i, 0)), _row(BN, skip_d)]
    args = [parts, skip_x]
    if has_sw:
        in_specs += [_full((skip_d, 128)), _full((1, 128))]
        args += [skip_w, skip_b]
    if has_l:
        in_specs += [_row(BN, 128), _full((128, 128)), _full((1, 128))]
        args += [x1, slw, slb]
    in_specs += [_full((1, 128)), _full((1, 128)), _full((8, 128)),
                 _full((128, ds_n)), _full((128, dd_n))]
    args += [lnw, lnb, b8, wsn, wdn]
    return pl.pallas_call(
        body, grid=(NN // BN,), in_specs=in_specs,
        out_specs=[_row(BN, 128), _row(BN, ds_n), _row(BN, dd_n)],
        out_shape=[jax.ShapeDtypeStruct((NN, 128), jnp.float32),
                   jax.ShapeDtypeStruct((NN, ds_n), jnp.float32),
                   jax.ShapeDtypeStruct((NN, dd_n), jnp.float32)],
    )(*args)


def _edge_head(gso, gdo, e2, ee, b1, w2, b2, w3, b3, w4p, b4p):
    def body(gs_r, gd_r, e_r, eer, b1r, w2r, b2r, w3r, b3r, w4r, b4r, l_o, p_o):
        g = jnp.maximum(gs_r[...] + gd_r[...] + _mm(e_r[...], eer[...]) + b1r[...], 0.0)
        g = jnp.maximum(_mm(g, w2r[...]) + b2r[...], 0.0)
        g = jnp.maximum(_mm(g, w3r[...]) + b3r[...], 0.0)
        l8 = _mm(g, w4r[...]) + b4r[...]
        l_o[...] = l8
        p_o[...] = 1.0 / (1.0 + jnp.exp(-l8))

    return pl.pallas_call(
        body, grid=(NE // BEB,),
        in_specs=[_row(BEB, 128), _row(BEB, 128), _row(BEB, 32),
                  _full((32, 128)), _full((1, 128)), _full((128, 64)), _full((1, 64)),
                  _full((64, 32)), _full((1, 32)), _full((32, 8)), _full((1, 8))],
        out_specs=[_row(BEB, 8), _row(BEB, 8)],
        out_shape=[jax.ShapeDtypeStruct((NE, 8), jnp.float32),
                   jax.ShapeDtypeStruct((NE, 8), jnp.float32)],
    )(gso, gdo, e2, ee, b1, w2, b2, w3, b3, w4p, b4p)


def _node_head(xo, w1, b1, w2, b2, w3, b3, w4p, b4p):
    def body(x_r, w1r, b1r, w2r, b2r, w3r, b3r, w4r, b4r, l_o, p_o):
        h = jnp.maximum(_mm(x_r[...], w1r[...]) + b1r[...], 0.0)
        h = jnp.maximum(_mm(h, w2r[...]) + b2r[...], 0.0)
        h = jnp.maximum(_mm(h, w3r[...]) + b3r[...], 0.0)
        l8 = _mm(h, w4r[...]) + b4r[...]
        l_o[...] = l8
        z = jnp.exp(l8 - jnp.max(l8, -1, keepdims=True))
        p_o[...] = z / jnp.sum(z, -1, keepdims=True)

    return pl.pallas_call(
        body, grid=(NN // BN,),
        in_specs=[_row(BN, 128),
                  _full((128, 64)), _full((1, 64)), _full((64, 32)), _full((1, 32)),
                  _full((32, 16)), _full((1, 16)), _full((16, 8)), _full((1, 8))],
        out_specs=[_row(BN, 8), _row(BN, 8)],
        out_shape=[jax.ShapeDtypeStruct((NN, 8), jnp.float32),
                   jax.ShapeDtypeStruct((NN, 8), jnp.float32)],
    )(xo, w1, b1, w2, b2, w3, b3, w4p, b4p)


def _r2(b):
    return b.reshape(1, -1)


def _conv_w(cp, nd):
    a1w = cp['a1'][0]
    ai, aj, ae = a1w[:nd], a1w[nd:2 * nd], a1w[2 * nd:]
    m1w = cp['m1'][0]
    mx, me = m1w[:nd], m1w[nd:]
    a2p = jnp.pad(cp['a2'][0], ((0, 0), (0, 8 - HEADS)))
    ba2 = jnp.pad(cp['a2'][1], (0, 8 - HEADS))
    b8 = jnp.repeat(jnp.eye(HEADS, dtype=jnp.float32), HID, axis=1)
    b8 = jnp.pad(b8, ((0, 8 - HEADS), (0, 0)))
    W = {'ae': ae, 'ba1': _r2(cp['a1'][1]), 'a2p': a2p, 'ba2': _r2(ba2),
         'me': me, 'bm1': _r2(cp['m1'][1]), 'm2': cp['m2'][0],
         'bm2': _r2(cp['m2'][1]), 'b8': b8}
    return W, ai, aj, mx


def kernel(x_in, edge_index, edge_attr, params):
    P = params
    src = edge_index[0]
    dst = edge_index[1]
    dst3 = dst.reshape(NW, NCHUNKS, KS)

    W1, ai1, aj1, mx1 = _conv_w(P['conv1'], HID)
    W2, ai2, aj2, mx2 = _conv_w(P['conv2'], OUT)
    W3, ai3, aj3, mx3 = _conv_w(P['conv3'], OUT)
    u1w = P['eup1']['w1'][0]
    W2.update({'ue': u1w[2 * OUT:], 'bu1': _r2(P['eup1']['w1'][1]),
               'uw2': P['eup1']['w2'][0], 'bu2': _r2(P['eup1']['w2'][1]),
               'lnw': _r2(P['eup1']['ln'][0]), 'lnb': _r2(P['eup1']['ln'][1])})
    u2w = P['eup2']['w1'][0]
    W3.update({'ue': u2w[2 * OUT:], 'bu1': _r2(P['eup2']['w1'][1]),
               'uw2': P['eup2']['w2'][0], 'bu2': _r2(P['eup2']['w2'][1]),
               'lnw': _r2(P['eup2']['ln'][0]), 'lnb': _r2(P['eup2']['ln'][1])})

    ws0 = jnp.concatenate([mx1, aj1], 1)
    wd0 = ai1
    ws1 = jnp.concatenate([mx2, aj2, u1w[OUT:2 * OUT]], 1)
    wd1 = jnp.concatenate([ai2, u1w[:OUT]], 1)
    ws2 = jnp.concatenate([mx3, aj3, u2w[OUT:2 * OUT]], 1)
    wd2 = jnp.concatenate([ai3, u2w[:OUT]], 1)
    ehw = P['eh1'][0]
    wso, wdo = ehw[:OUT], ehw[OUT:2 * OUT]

    x0, ts0, td0 = _node_encode(
        x_in, _r2(P['dummy']), P['ne1'][0], _r2(P['ne1'][1]),
        P['ne2'][0], _r2(P['ne2'][1]), ws0, wd0)
    e0 = _edge_encode(edge_attr, P['ee1'][0], _r2(P['ee1'][1]),
                      P['ee2'][0], _r2(P['ee2'][1]))

    gs, gd = _sc_gather(ts0, td0, src, dst, 160, 32)
    wex, _ = _edge_stage(gs, gd, e0, W1, False)
    parts = _sc_scatter(wex, dst3)
    x1, ts1, td1 = _combine(parts, x0, P['skip0'][0], _r2(P['skip0'][1]),
                            _r2(P['ln1'][0]), _r2(P['ln1'][1]), W1['b8'], ws1, wd1)

    gs, gd = _sc_gather(ts1, td1, src, dst, 192, 64)
    wex, e1 = _edge_stage(gs, gd, e0, W2, True)
    parts = _sc_scatter(wex, dst3)
    x2, ts2, td2 = _combine(parts, x1, None, None,
                            _r2(P['ln2'][0]), _r2(P['ln2'][1]), W1['b8'], ws2, wd2)

    gs, gd = _sc_gather(ts2, td2, src, dst, 192, 64)
    wex, e2 = _edge_stage(gs, gd, e1, W3, True)
    parts = _sc_scatter(wex, dst3)
    xo, tso, tdo = _combine(parts, x2, None, None,
                            _r2(P['ln3'][0]), _r2(P['ln3'][1]), W1['b8'], wso, wdo,
                            x1=x1, slw=P['skipL'][0], slb=_r2(P['skipL'][1]))

    gso, gdo = _sc_gather(tso, tdo, src, dst, 128, 128)
    el8, ep8 = _edge_head(gso, gdo, e2, ehw[2 * OUT:], _r2(P['eh1'][1]),
                          P['eh2'][0], _r2(P['eh2'][1]),
                          P['eh3'][0], _r2(P['eh3'][1]),
                          jnp.pad(P['eh4'][0], ((0, 0), (0, 7))),
                          _r2(jnp.pad(P['eh4'][1], (0, 7))))

    nb4 = jnp.pad(P['nh4'][1], (0, 1), constant_values=-1e30)
    nl8, np8 = _node_head(xo, P['nh1'][0], _r2(P['nh1'][1]),
                          P['nh2'][0], _r2(P['nh2'][1]),
                          P['nh3'][0], _r2(P['nh3'][1]),
                          jnp.pad(P['nh4'][0], ((0, 0), (0, 1))), _r2(nb4))

    return (nl8[:, :7], el8[:, :1], np8[:, :7], ep8[:, :1])


# speedup vs baseline: 4.0866x; 1.3682x over previous
import functools

import jax
import jax.numpy as jnp
from jax import lax
from jax.experimental import pallas as pl
from jax.experimental.pallas import tpu as pltpu
from jax.experimental.pallas import tpu_sc as plsc

NN = 10000
NE = 320000
HID = 32
OUT = 128
HEADS = 4

NC, NS = 2, 16
NW = NC * NS
EPW = NE // NW
KS = 40
NCHUNKS = EPW // KS
NB = 5
ZR = 40

BN = 2000
BEB = 2000


def _row(bs, d):
    return pl.BlockSpec((bs, d), lambda i: (i, 0))


def _full(shape):
    nd = len(shape)
    return pl.BlockSpec(shape, lambda i: (0,) * nd)


def _ln(x, w, b, eps=1e-5):
    m = jnp.mean(x, -1, keepdims=True)
    v = jnp.mean((x - m) * (x - m), -1, keepdims=True)
    return (x - m) * jax.lax.rsqrt(v + eps) * w + b


def _mm(x, w):
    return jnp.dot(x, w, preferred_element_type=jnp.float32)


def _sc_gather_multi(tables, sels, src, dst):
    G = len(tables)
    mesh = plsc.VectorSubcoreMesh(core_axis_name="c", subcore_axis_name="s")
    scratch = [pltpu.VMEM((EPW,), jnp.int32), pltpu.VMEM((EPW,), jnp.int32)]
    scratch += [pltpu.VMEM((KS, 128), jnp.float32) for _ in range(G * NB)]
    scratch += [pltpu.SemaphoreType.DMA for _ in range(2 * G * NB)]

    @functools.partial(
        pl.kernel, mesh=mesh,
        out_type=[jax.ShapeDtypeStruct((NE, 128), jnp.float32) for _ in range(G)],
        compiler_params=pltpu.CompilerParams(use_tc_tiling_on_sc=True),
        scratch_types=scratch)
    def k(*refs):
        tbls = refs[:G]
        src_hbm, dst_hbm = refs[G], refs[G + 1]
        outs = refs[G + 2:2 * G + 2]
        sc = refs[2 * G + 2:]
        idx_s, idx_d = sc[0], sc[1]
        bufs = [sc[2 + g * NB:2 + (g + 1) * NB] for g in range(G)]
        gsem = [sc[2 + G * NB + g * NB:2 + G * NB + (g + 1) * NB] for g in range(G)]
        ssem = [sc[2 + 2 * G * NB + g * NB:2 + 2 * G * NB + (g + 1) * NB]
                for g in range(G)]
        wid = lax.axis_index("s") * NC + lax.axis_index("c")
        base = wid * EPW
        pltpu.sync_copy(src_hbm.at[pl.ds(base, EPW)], idx_s)
        pltpu.sync_copy(dst_hbm.at[pl.ds(base, EPW)], idx_d)
        idx = [idx_s if s == 0 else idx_d for s in sels]

        def rnd(r, _):
            off = r * (NB * KS)
            for b in range(NB):
                co = off + b * KS
                for g in range(G):
                    pltpu.async_copy(tbls[g].at[idx[g].at[pl.ds(co, KS)]],
                                     bufs[g][b], gsem[g][b])
            for b in range(NB):
                co = off + b * KS
                for g in range(G):
                    pltpu.make_async_copy(tbls[g].at[idx[g].at[pl.ds(co, KS)]],
                                          bufs[g][b], gsem[g][b]).wait()
            for b in range(NB):
                co = off + b * KS
                for g in range(G):
                    pltpu.async_copy(bufs[g][b], outs[g].at[pl.ds(base + co, KS)],
                                     ssem[g][b])
            for b in range(NB):
                co = off + b * KS
                for g in range(G):
                    pltpu.make_async_copy(bufs[g][b], outs[g].at[pl.ds(base + co, KS)],
                                          ssem[g][b]).wait()
            return 0

        lax.fori_loop(0, NCHUNKS // NB, rnd, 0)

    return k(*tables, src, dst)


def _sc_scatter(wex, dst, width, tiled):
    mesh = plsc.VectorSubcoreMesh(core_axis_name="c", subcore_axis_name="s")
    scratch = [pltpu.VMEM((KS,), jnp.int32) for _ in range(NB)]
    scratch += [pltpu.VMEM((KS, width), jnp.float32) for _ in range(NB)]
    scratch += [pltpu.VMEM((ZR, width), jnp.float32)]
    scratch += [pltpu.VMEM_SHARED((NN, width), jnp.float32)]
    scratch += [pltpu.SemaphoreType.DMA for _ in range(3 * NB)]

    @functools.partial(
        pl.kernel, mesh=mesh,
        out_type=jax.ShapeDtypeStruct((NC, NN, width), jnp.float32),
        compiler_params=pltpu.CompilerParams(use_tc_tiling_on_sc=tiled),
        scratch_types=scratch)
    def k(wex_hbm, dst_hbm, out_hbm, *sc):
        idxb = sc[:NB]
        buf = sc[NB:2 * NB]
        zbuf = sc[2 * NB]
        acc = sc[2 * NB + 1]
        isem = sc[2 * NB + 2:2 * NB + 2 + NB]
        lsem = sc[2 * NB + 2 + NB:2 * NB + 2 + 2 * NB]
        ssem = sc[2 * NB + 2 + 2 * NB:2 * NB + 2 + 3 * NB]
        cid = lax.axis_index("c")
        sid = lax.axis_index("s")
        wid = sid * NC + cid

        def zrow(i, _):
            for j in range(width // 16):
                zbuf[i, pl.ds(j * 16, 16)] = jnp.zeros((16,), jnp.float32)
            return 0

        lax.fori_loop(0, ZR, zrow, 0)

        @pl.when(sid < 10)
        def _():
            for c in range(1000 // ZR):
                pltpu.sync_copy(zbuf, acc.at[pl.ds(sid * 1000 + c * ZR, ZR)])

        plsc.subcore_barrier()

        base = wid * EPW

        def rnd(r, _):
            for b in range(NB):
                j = r * NB + b
                pltpu.async_copy(dst_hbm.at[pl.ds(base + j * KS, KS)], idxb[b], isem[b])
                pltpu.async_copy(wex_hbm.at[pl.ds(base + j * KS, KS)], buf[b], lsem[b])
            for b in range(NB):
                j = r * NB + b
                pltpu.make_async_copy(dst_hbm.at[pl.ds(base + j * KS, KS)], idxb[b], isem[b]).wait()
                pltpu.make_async_copy(wex_hbm.at[pl.ds(base + j * KS, KS)], buf[b], lsem[b]).wait()
            for b in range(NB):
                pltpu.async_copy(buf[b], acc.at[idxb[b]], ssem[b], add=True)
            for b in range(NB):
                pltpu.make_async_copy(buf[b], acc.at[idxb[b]], ssem[b]).wait()
            return 0

        lax.fori_loop(0, NCHUNKS // NB, rnd, 0)
        plsc.subcore_barrier()

        @pl.when(sid < 10)
        def _():
            pltpu.sync_copy(acc.at[pl.ds(sid * 1000, 1000)],
                            out_hbm.at[cid, pl.ds(sid * 1000, 1000)])

    return k(wex, dst)


def _node_encode(x_in, dummy, ne1w, ne1b, ne2w, ne2b, wm0, wa0):
    def body(x_ref, dm, w1, b1, w2, b2, wmr, war, x0_o, tm_o, ta_o):
        x = x_ref[...]
        inv = x[:, 0:1] == -999.0
        xc = jnp.where(inv, dm[...], x)
        h = jnp.maximum(_mm(xc, w1[...]) + b1[...], 0.0)
        x0 = _mm(h, w2[...]) + b2[...]
        x0_o[...] = x0
        tm_o[...] = _mm(x0, wmr[...])
        ta_o[...] = _mm(x0, war[...])

    grid = (NN // BN,)
    return pl.pallas_call(
        body, grid=grid,
        in_specs=[_row(BN, 128), _full((1, 128)), _full((128, 32)), _full((1, 32)),
                  _full((32, 32)), _full((1, 32)), _full((32, 128)), _full((32, 128))],
        out_specs=[_row(BN, 32), _row(BN, 128), _row(BN, 128)],
        out_shape=[jax.ShapeDtypeStruct((NN, 32), jnp.float32),
                   jax.ShapeDtypeStruct((NN, 128), jnp.float32),
                   jax.ShapeDtypeStruct((NN, 128), jnp.float32)],
    )(x_in, dummy, ne1w, ne1b, ne2w, ne2b, wm0, wa0)


def _edge_encode(ea, w1, b1, w2, b2):
    def body(e_ref, w1r, b1r, w2r, b2r, o):
        h = jnp.maximum(_mm(e_ref[...], w1r[...]) + b1r[...], 0.0)
        o[...] = _mm(h, w2r[...]) + b2r[...]

    return pl.pallas_call(
        body, grid=(NE // BEB,),
        in_specs=[_row(BEB, 16), _full((16, 32)), _full((1, 32)),
                  _full((32, 32)), _full((1, 32))],
        out_specs=_row(BEB, 32),
        out_shape=jax.ShapeDtypeStruct((NE, 32), jnp.float32),
    )(ea, w1, b1, w2, b2)


def _edge_stage(gmsg, gaus, gaud, e_prev, W, has_update):
    def body(*refs):
        if has_update:
            (gm_r, gs_r, gd_r, ep_r, ue, bu1, uw2, bu2, lnw, lnb,
             ae, ba1, a2p, ba2, me, bm1, m2, bm2, b8, wa_o, wb_o, e_o) = refs
        else:
            (gm_r, gs_r, gd_r, ep_r,
             ae, ba1, a2p, ba2, me, bm1, m2, bm2, b8, wa_o, wb_o) = refs
        gs_v = gs_r[...]
        gd_v = gd_r[...]
        ep = ep_r[...]
        if has_update:
            uh = jnp.maximum(gd_v[:, 96:128] + gs_v[:, 32:64]
                             + _mm(ep, ue[...]) + bu1[...], 0.0)
            e = _ln(ep + _mm(uh, uw2[...]) + bu2[...], lnw[...], lnb[...])
            e_o[...] = e
            att_d = gd_v[:, 64:96]
        else:
            e = ep
            att_d = gd_v[:, 32:64]
        ah = jnp.maximum(att_d + gs_v[:, 0:32] + _mm(e, ae[...]) + ba1[...], 0.0)
        ex8 = jnp.exp(_mm(ah, a2p[...]) + ba2[...])
        mh = jnp.maximum(gm_r[...] + _mm(e, me[...]) + bm1[...], 0.0)
        m = _mm(mh, m2[...]) + bm2[...]
        wa_o[...] = m * _mm(ex8, b8[...])
        wb_o[:, 0:8] = ex8
        wb_o[:, 8:16] = jnp.zeros((BEB, 8), jnp.float32)

    in_specs = [_row(BEB, 128), _row(BEB, 128), _row(BEB, 128), _row(BEB, 32)]
    args = [gmsg, gaus, gaud, e_prev]
    if has_update:
        in_specs += [_full((32, 32)), _full((1, 32)), _full((32, 32)), _full((1, 32)),
                     _full((1, 32)), _full((1, 32))]
        args += [W['ue'], W['bu1'], W['uw2'], W['bu2'], W['lnw'], W['lnb']]
    in_specs += [_full((32, 32)), _full((1, 32)), _full((32, 8)), _full((1, 8)),
                 _full((32, 128)), _full((1, 128)), _full((128, 128)), _full((1, 128)),
                 _full((8, 128))]
    args += [W['ae'], W['ba1'], W['a2p'], W['ba2'], W['me'], W['bm1'],
             W['m2'], W['bm2'], W['b8']]
    out_specs = [_row(BEB, 128), _row(BEB, 16)]
    out_shape = [jax.ShapeDtypeStruct((NE, 128), jnp.float32),
                 jax.ShapeDtypeStruct((NE, 16), jnp.float32)]
    if has_update:
        out_specs += [_row(BEB, 32)]
        out_shape += [jax.ShapeDtypeStruct((NE, 32), jnp.float32)]
    res = pl.pallas_call(body, grid=(NE // BEB,), in_specs=in_specs,
                         out_specs=out_specs, out_shape=out_shape)(*args)
    return res if has_update else (res[0], res[1], None)


def _combine(pa, pb, skip_x, skip_w, skip_b, lnw, lnb, b8, wmn, wan,
             x1=None, slw=None, slb=None):
    has_sw = skip_w is not None
    has_l = x1 is not None
    skip_d = skip_x.shape[1]

    def body(*refs):
        i = 0
        pa_r = refs[i]; i += 1
        pb_r = refs[i]; i += 1
        sk_r = refs[i]; i += 1
        if has_sw:
            skw = refs[i]; i += 1
            skb = refs[i]; i += 1
        if has_l:
            x1_r = refs[i]; i += 1
            slw_r = refs[i]; i += 1
            slb_r = refs[i]; i += 1
        lnw_r = refs[i]; i += 1
        lnb_r = refs[i]; i += 1
        b8_r = refs[i]; i += 1
        wmn_r = refs[i]; i += 1
        wan_r = refs[i]; i += 1
        x_o, tm_o, ta_o = refs[i], refs[i + 1], refs[i + 2]

        Sb = pb_r[0] + pb_r[1]
        den = _mm(Sb[:, 0:8], b8_r[...]) + 1e-16
        agg = (pa_r[0] + pa_r[1]) / den
        if has_sw:
            sk = _mm(sk_r[...], skw[...]) + skb[...]
        else:
            sk = sk_r[...]
        x = jnp.maximum(_ln(agg + sk, lnw_r[...], lnb_r[...]), 0.0)
        if has_l:
            x = x + _mm(x1_r[...], slw_r[...]) + slb_r[...]
        x_o[...] = x
        tm_o[...] = _mm(x, wmn_r[...])
        ta_o[...] = _mm(x, wan_r[...])

    in_specs = [pl.BlockSpec((2, BN, 128), lambda i: (0, i, 0)),
                pl.BlockSpec((2, BN, 16), lambda i: (0, i, 0)),
                _row(BN, skip_d)]
    args = [pa, pb, skip_x]
    if has_sw:
        in_specs += [_full((skip_d, 128)), _full((1, 128))]
        args += [skip_w, skip_b]
    if has_l:
        in_specs += [_row(BN, 128), _full((128, 128)), _full((1, 128))]
        args += [x1, slw, slb]
    in_specs += [_full((1, 128)), _full((1, 128)), _full((8, 128)),
                 _full((wmn.shape[0], 128)), _full((wan.shape[0], 128))]
    args += [lnw, lnb, b8, wmn, wan]
    return pl.pallas_call(
        body, grid=(NN // BN,), in_specs=in_specs,
        out_specs=[_row(BN, 128), _row(BN, 128), _row(BN, 128)],
        out_shape=[jax.ShapeDtypeStruct((NN, 128), jnp.float32),
                   jax.ShapeDtypeStruct((NN, 128), jnp.float32),
                   jax.ShapeDtypeStruct((NN, 128), jnp.float32)],
    )(*args)


def _edge_head(gso, gdo, e2, ee, b1, w2, b2, w3, b3, w4p, b4p):
    def body(gs_r, gd_r, e_r, eer, b1r, w2r, b2r, w3r, b3r, w4r, b4r, l_o, p_o):
        g = jnp.maximum(gs_r[...] + gd_r[...] + _mm(e_r[...], eer[...]) + b1r[...], 0.0)
        g = jnp.maximum(_mm(g, w2r[...]) + b2r[...], 0.0)
        g = jnp.maximum(_mm(g, w3r[...]) + b3r[...], 0.0)
        l8 = _mm(g, w4r[...]) + b4r[...]
        l_o[...] = l8
        p_o[...] = 1.0 / (1.0 + jnp.exp(-l8))

    return pl.pallas_call(
        body, grid=(NE // BEB,),
        in_specs=[_row(BEB, 128), _row(BEB, 128), _row(BEB, 32),
                  _full((32, 128)), _full((1, 128)), _full((128, 64)), _full((1, 64)),
                  _full((64, 32)), _full((1, 32)), _full((32, 8)), _full((1, 8))],
        out_specs=[_row(BEB, 8), _row(BEB, 8)],
        out_shape=[jax.ShapeDtypeStruct((NE, 8), jnp.float32),
                   jax.ShapeDtypeStruct((NE, 8), jnp.float32)],
    )(gso, gdo, e2, ee, b1, w2, b2, w3, b3, w4p, b4p)


def _node_head(xo, w1, b1, w2, b2, w3, b3, w4p, b4p):
    def body(x_r, w1r, b1r, w2r, b2r, w3r, b3r, w4r, b4r, l_o, p_o):
        h = jnp.maximum(_mm(x_r[...], w1r[...]) + b1r[...], 0.0)
        h = jnp.maximum(_mm(h, w2r[...]) + b2r[...], 0.0)
        h = jnp.maximum(_mm(h, w3r[...]) + b3r[...], 0.0)
        l8 = _mm(h, w4r[...]) + b4r[...]
        l_o[...] = l8
        z = jnp.exp(l8 - jnp.max(l8, -1, keepdims=True))
        p_o[...] = z / jnp.sum(z, -1, keepdims=True)

    return pl.pallas_call(
        body, grid=(NN // BN,),
        in_specs=[_row(BN, 128),
                  _full((128, 64)), _full((1, 64)), _full((64, 32)), _full((1, 32)),
                  _full((32, 16)), _full((1, 16)), _full((16, 8)), _full((1, 8))],
        out_specs=[_row(BN, 8), _row(BN, 8)],
        out_shape=[jax.ShapeDtypeStruct((NN, 8), jnp.float32),
                   jax.ShapeDtypeStruct((NN, 8), jnp.float32)],
    )(xo, w1, b1, w2, b2, w3, b3, w4p, b4p)


def _r2(b):
    return b.reshape(1, -1)


def _conv_w(cp, nd):
    a1w = cp['a1'][0]
    ai, aj, ae = a1w[:nd], a1w[nd:2 * nd], a1w[2 * nd:]
    m1w = cp['m1'][0]
    mx, me = m1w[:nd], m1w[nd:]
    a2p = jnp.pad(cp['a2'][0], ((0, 0), (0, 8 - HEADS)))
    ba2 = jnp.pad(cp['a2'][1], (0, 8 - HEADS))
    b8 = jnp.repeat(jnp.eye(HEADS, dtype=jnp.float32), HID, axis=1)
    b8 = jnp.pad(b8, ((0, 8 - HEADS), (0, 0)))
    W = {'ae': ae, 'ba1': _r2(cp['a1'][1]), 'a2p': a2p, 'ba2': _r2(ba2),
         'me': me, 'bm1': _r2(cp['m1'][1]), 'm2': cp['m2'][0],
         'bm2': _r2(cp['m2'][1]), 'b8': b8}
    return W, ai, aj, mx


def kernel(x_in, edge_index, edge_attr, params):
    P = params
    src = edge_index[0]
    dst = edge_index[1]

    W1, ai1, aj1, mx1 = _conv_w(P['conv1'], HID)
    W2, ai2, aj2, mx2 = _conv_w(P['conv2'], OUT)
    W3, ai3, aj3, mx3 = _conv_w(P['conv3'], OUT)
    u1w = P['eup1']['w1'][0]
    W2.update({'ue': u1w[2 * OUT:], 'bu1': _r2(P['eup1']['w1'][1]),
               'uw2': P['eup1']['w2'][0], 'bu2': _r2(P['eup1']['w2'][1]),
               'lnw': _r2(P['eup1']['ln'][0]), 'lnb': _r2(P['eup1']['ln'][1])})
    u2w = P['eup2']['w1'][0]
    W3.update({'ue': u2w[2 * OUT:], 'bu1': _r2(P['eup2']['w1'][1]),
               'uw2': P['eup2']['w2'][0], 'bu2': _r2(P['eup2']['w2'][1]),
               'lnw': _r2(P['eup2']['ln'][0]), 'lnb': _r2(P['eup2']['ln'][1])})

    wm0 = mx1
    wa0 = jnp.concatenate([aj1, ai1, jnp.zeros((HID, 64), jnp.float32)], 1)
    wa1 = jnp.concatenate([aj2, u1w[OUT:2 * OUT], ai2, u1w[:OUT]], 1)
    wa2 = jnp.concatenate([aj3, u2w[OUT:2 * OUT], ai3, u2w[:OUT]], 1)
    ehw = P['eh1'][0]
    wso, wdo = ehw[:OUT], ehw[OUT:2 * OUT]

    x0, tm0, ta0 = _node_encode(
        x_in, _r2(P['dummy']), P['ne1'][0], _r2(P['ne1'][1]),
        P['ne2'][0], _r2(P['ne2'][1]), wm0, wa0)
    e0 = _edge_encode(edge_attr, P['ee1'][0], _r2(P['ee1'][1]),
                      P['ee2'][0], _r2(P['ee2'][1]))

    gm, gas, gad = _sc_gather_multi([tm0, ta0, ta0], [0, 0, 1], src, dst)
    wexa, wexb, _ = _edge_stage(gm, gas, gad, e0, W1, False)
    pa = _sc_scatter(wexa, dst, 128, True)
    pb = _sc_scatter(wexb, dst, 16, False)
    x1, tm1, ta1 = _combine(pa, pb, x0, P['skip0'][0], _r2(P['skip0'][1]),
                            _r2(P['ln1'][0]), _r2(P['ln1'][1]), W1['b8'], mx2, wa1)

    gm, gas, gad = _sc_gather_multi([tm1, ta1, ta1], [0, 0, 1], src, dst)
    wexa, wexb, e1 = _edge_stage(gm, gas, gad, e0, W2, True)
    pa = _sc_scatter(wexa, dst, 128, True)
    pb = _sc_scatter(wexb, dst, 16, False)
    x2, tm2, ta2 = _combine(pa, pb, x1, None, None,
                            _r2(P['ln2'][0]), _r2(P['ln2'][1]), W1['b8'], mx3, wa2)

    gm, gas, gad = _sc_gather_multi([tm2, ta2, ta2], [0, 0, 1], src, dst)
    wexa, wexb, e2 = _edge_stage(gm, gas, gad, e1, W3, True)
    pa = _sc_scatter(wexa, dst, 128, True)
    pb = _sc_scatter(wexb, dst, 16, False)
    xo, tso, tdo = _combine(pa, pb, x2, None, None,
                            _r2(P['ln3'][0]), _r2(P['ln3'][1]), W1['b8'], wso, wdo,
                            x1=x1, slw=P['skipL'][0], slb=_r2(P['skipL'][1]))

    gso, gdo = _sc_gather_multi([tso, tdo], [0, 1], src, dst)
    el8, ep8 = _edge_head(gso, gdo, e2, ehw[2 * OUT:], _r2(P['eh1'][1]),
                          P['eh2'][0], _r2(P['eh2'][1]),
                          P['eh3'][0], _r2(P['eh3'][1]),
                          jnp.pad(P['eh4'][0], ((0, 0), (0, 7))),
                          _r2(jnp.pad(P['eh4'][1], (0, 7))))

    nb4 = jnp.pad(P['nh4'][1], (0, 1), constant_values=-1e30)
    nl8, np8 = _node_head(xo, P['nh1'][0], _r2(P['nh1'][1]),
                          P['nh2'][0], _r2(P['nh2'][1]),
                          P['nh3'][0], _r2(P['nh3'][1]),
                          jnp.pad(P['nh4'][0], ((0, 0), (0, 1))), _r2(nb4))

    return (nl8[:, :7], el8[:, :1], np8[:, :7], ep8[:, :1])


# speedup vs baseline: 4.1808x; 1.0231x over previous
import functools

import jax
import jax.numpy as jnp
from jax import lax
from jax.experimental import pallas as pl
from jax.experimental.pallas import tpu as pltpu
from jax.experimental.pallas import tpu_sc as plsc

NN = 10000
NE = 320000
HID = 32
OUT = 128
HEADS = 4

NC, NS = 2, 16
NW = NC * NS
EPW = NE // NW
KS = 40
NCHUNKS = EPW // KS
NB = 5
ZR = 40

BN = 2000
BEB = 2000


def _row(bs, d):
    return pl.BlockSpec((bs, d), lambda i: (i, 0))


def _full(shape):
    nd = len(shape)
    return pl.BlockSpec(shape, lambda i: (0,) * nd)


def _ln(x, w, b, eps=1e-5):
    m = jnp.mean(x, -1, keepdims=True)
    v = jnp.mean((x - m) * (x - m), -1, keepdims=True)
    return (x - m) * jax.lax.rsqrt(v + eps) * w + b


def _mm(x, w):
    return jnp.dot(x, w, preferred_element_type=jnp.float32)


def _sc_gather_multi(tables, sels, src, dst):
    G = len(tables)
    mesh = plsc.VectorSubcoreMesh(core_axis_name="c", subcore_axis_name="s")
    scratch = [pltpu.VMEM((EPW,), jnp.int32), pltpu.VMEM((EPW,), jnp.int32)]
    scratch += [pltpu.VMEM((KS, 128), jnp.float32) for _ in range(G * NB)]
    scratch += [pltpu.SemaphoreType.DMA for _ in range(2 * G * NB)]

    @functools.partial(
        pl.kernel, mesh=mesh,
        out_type=[jax.ShapeDtypeStruct((NE, 128), jnp.float32) for _ in range(G)],
        compiler_params=pltpu.CompilerParams(use_tc_tiling_on_sc=True),
        scratch_types=scratch)
    def k(*refs):
        tbls = refs[:G]
        src_hbm, dst_hbm = refs[G], refs[G + 1]
        outs = refs[G + 2:2 * G + 2]
        sc = refs[2 * G + 2:]
        idx_s, idx_d = sc[0], sc[1]
        bufs = [sc[2 + g * NB:2 + (g + 1) * NB] for g in range(G)]
        gsem = [sc[2 + G * NB + g * NB:2 + G * NB + (g + 1) * NB] for g in range(G)]
        ssem = [sc[2 + 2 * G * NB + g * NB:2 + 2 * G * NB + (g + 1) * NB]
                for g in range(G)]
        wid = lax.axis_index("s") * NC + lax.axis_index("c")
        base = wid * EPW
        pltpu.sync_copy(src_hbm.at[pl.ds(base, EPW)], idx_s)
        pltpu.sync_copy(dst_hbm.at[pl.ds(base, EPW)], idx_d)
        idx = [idx_s if s == 0 else idx_d for s in sels]

        def rnd(r, _):
            off = r * (NB * KS)
            for b in range(NB):
                co = off + b * KS
                for g in range(G):
                    pltpu.async_copy(tbls[g].at[idx[g].at[pl.ds(co, KS)]],
                                     bufs[g][b], gsem[g][b])
            for b in range(NB):
                co = off + b * KS
                for g in range(G):
                    pltpu.make_async_copy(tbls[g].at[idx[g].at[pl.ds(co, KS)]],
                                          bufs[g][b], gsem[g][b]).wait()
            for b in range(NB):
                co = off + b * KS
                for g in range(G):
                    pltpu.async_copy(bufs[g][b], outs[g].at[pl.ds(base + co, KS)],
                                     ssem[g][b])
            for b in range(NB):
                co = off + b * KS
                for g in range(G):
                    pltpu.make_async_copy(bufs[g][b], outs[g].at[pl.ds(base + co, KS)],
                                          ssem[g][b]).wait()
            return 0

        lax.fori_loop(0, NCHUNKS // NB, rnd, 0)

    return k(*tables, src, dst)


def _sc_scatter(wex, dst, width, tiled):
    mesh = plsc.VectorSubcoreMesh(core_axis_name="c", subcore_axis_name="s")
    scratch = [pltpu.VMEM((KS,), jnp.int32) for _ in range(NB)]
    scratch += [pltpu.VMEM((KS, width), jnp.float32) for _ in range(NB)]
    scratch += [pltpu.VMEM((ZR, width), jnp.float32)]
    scratch += [pltpu.VMEM_SHARED((NN, width), jnp.float32)]
    scratch += [pltpu.SemaphoreType.DMA for _ in range(3 * NB)]

    @functools.partial(
        pl.kernel, mesh=mesh,
        out_type=jax.ShapeDtypeStruct((NC, NN, width), jnp.float32),
        compiler_params=pltpu.CompilerParams(use_tc_tiling_on_sc=tiled),
        scratch_types=scratch)
    def k(wex_hbm, dst_hbm, out_hbm, *sc):
        idxb = sc[:NB]
        buf = sc[NB:2 * NB]
        zbuf = sc[2 * NB]
        acc = sc[2 * NB + 1]
        isem = sc[2 * NB + 2:2 * NB + 2 + NB]
        lsem = sc[2 * NB + 2 + NB:2 * NB + 2 + 2 * NB]
        ssem = sc[2 * NB + 2 + 2 * NB:2 * NB + 2 + 3 * NB]
        cid = lax.axis_index("c")
        sid = lax.axis_index("s")
        wid = sid * NC + cid

        def zrow(i, _):
            for j in range(width // 16):
                zbuf[i, pl.ds(j * 16, 16)] = jnp.zeros((16,), jnp.float32)
            return 0

        lax.fori_loop(0, ZR, zrow, 0)

        @pl.when(sid < 10)
        def _():
            for c in range(1000 // ZR):
                pltpu.sync_copy(zbuf, acc.at[pl.ds(sid * 1000 + c * ZR, ZR)])

        plsc.subcore_barrier()

        base = wid * EPW

        def rnd(r, _):
            for b in range(NB):
                j = r * NB + b
                pltpu.async_copy(dst_hbm.at[pl.ds(base + j * KS, KS)], idxb[b], isem[b])
                pltpu.async_copy(wex_hbm.at[pl.ds(base + j * KS, KS)], buf[b], lsem[b])
            for b in range(NB):
                j = r * NB + b
                pltpu.make_async_copy(dst_hbm.at[pl.ds(base + j * KS, KS)], idxb[b], isem[b]).wait()
                pltpu.make_async_copy(wex_hbm.at[pl.ds(base + j * KS, KS)], buf[b], lsem[b]).wait()
            for b in range(NB):
                pltpu.async_copy(buf[b], acc.at[idxb[b]], ssem[b], add=True)
            for b in range(NB):
                pltpu.make_async_copy(buf[b], acc.at[idxb[b]], ssem[b]).wait()
            return 0

        lax.fori_loop(0, NCHUNKS // NB, rnd, 0)
        plsc.subcore_barrier()

        @pl.when(sid < 10)
        def _():
            pltpu.sync_copy(acc.at[pl.ds(sid * 1000, 1000)],
                            out_hbm.at[cid, pl.ds(sid * 1000, 1000)])

    return k(wex, dst)


def _node_encode(x_in, dummy, ne1w, ne1b, ne2w, ne2b, wm0, wa0):
    def body(x_ref, dm, w1, b1, w2, b2, wmr, war, x0_o, tm_o, ta_o):
        x = x_ref[...]
        inv = x[:, 0:1] == -999.0
        xc = jnp.where(inv, dm[...], x)
        h = jnp.maximum(_mm(xc, w1[...]) + b1[...], 0.0)
        x0 = _mm(h, w2[...]) + b2[...]
        x0_o[...] = x0
        tm_o[...] = _mm(x0, wmr[...])
        ta_o[...] = _mm(x0, war[...])

    grid = (NN // BN,)
    return pl.pallas_call(
        body, grid=grid,
        in_specs=[_row(BN, 128), _full((1, 128)), _full((128, 32)), _full((1, 32)),
                  _full((32, 32)), _full((1, 32)), _full((32, 128)), _full((32, 128))],
        out_specs=[_row(BN, 32), _row(BN, 128), _row(BN, 128)],
        out_shape=[jax.ShapeDtypeStruct((NN, 32), jnp.float32),
                   jax.ShapeDtypeStruct((NN, 128), jnp.float32),
                   jax.ShapeDtypeStruct((NN, 128), jnp.float32)],
    )(x_in, dummy, ne1w, ne1b, ne2w, ne2b, wm0, wa0)


def _edge_encode(ea, w1, b1, w2, b2):
    def body(e_ref, w1r, b1r, w2r, b2r, o):
        h = jnp.maximum(_mm(e_ref[...], w1r[...]) + b1r[...], 0.0)
        o[...] = _mm(h, w2r[...]) + b2r[...]

    return pl.pallas_call(
        body, grid=(NE // BEB,),
        in_specs=[_row(BEB, 16), _full((16, 32)), _full((1, 32)),
                  _full((32, 32)), _full((1, 32))],
        out_specs=_row(BEB, 32),
        out_shape=jax.ShapeDtypeStruct((NE, 32), jnp.float32),
    )(ea, w1, b1, w2, b2)


def _edge_stage(gmsg, gaus, gaud, e_prev, W, has_update):
    def body(*refs):
        if has_update:
            (gm_r, gs_r, gd_r, ep_r, ue, bu1, uw2, bu2, lnw, lnb,
             ae, ba1, a2p, ba2, me, bm1, m2, bm2, b8, wa_o, wb_o) = refs
        else:
            (gm_r, gs_r, gd_r, ep_r,
             ae, ba1, a2p, ba2, me, bm1, m2, bm2, b8, wa_o, wb_o) = refs
        gs_v = gs_r[...]
        gd_v = gd_r[...]
        if has_update:
            ep = ep_r[...][:, 0:32] if ep_r.shape[1] == 128 else ep_r[...]
            uh = jnp.maximum(gd_v[:, 96:128] + gs_v[:, 32:64]
                             + _mm(ep, ue[...]) + bu1[...], 0.0)
            e = _ln(ep + _mm(uh, uw2[...]) + bu2[...], lnw[...], lnb[...])
            att_d = gd_v[:, 64:96]
        else:
            e = ep_r[...]
            att_d = gd_v[:, 32:64]
        ah = jnp.maximum(att_d + gs_v[:, 0:32] + _mm(e, ae[...]) + ba1[...], 0.0)
        ex8 = jnp.exp(_mm(ah, a2p[...]) + ba2[...])
        mh = jnp.maximum(gm_r[...] + _mm(e, me[...]) + bm1[...], 0.0)
        m = _mm(mh, m2[...]) + bm2[...]
        wa_o[...] = m * _mm(ex8, b8[...])
        if has_update:
            wb_o[:, 0:32] = e
        else:
            wb_o[:, 0:32] = jnp.zeros((BEB, 32), jnp.float32)
        wb_o[:, 32:40] = ex8
        wb_o[:, 40:128] = jnp.zeros((BEB, 88), jnp.float32)

    in_specs = [_row(BEB, 128), _row(BEB, 128), _row(BEB, 128),
                _row(BEB, e_prev.shape[1])]
    args = [gmsg, gaus, gaud, e_prev]
    if has_update:
        in_specs += [_full((32, 32)), _full((1, 32)), _full((32, 32)), _full((1, 32)),
                     _full((1, 32)), _full((1, 32))]
        args += [W['ue'], W['bu1'], W['uw2'], W['bu2'], W['lnw'], W['lnb']]
    in_specs += [_full((32, 32)), _full((1, 32)), _full((32, 8)), _full((1, 8)),
                 _full((32, 128)), _full((1, 128)), _full((128, 128)), _full((1, 128)),
                 _full((8, 128))]
    args += [W['ae'], W['ba1'], W['a2p'], W['ba2'], W['me'], W['bm1'],
             W['m2'], W['bm2'], W['b8']]
    out_specs = [_row(BEB, 128), _row(BEB, 128)]
    out_shape = [jax.ShapeDtypeStruct((NE, 128), jnp.float32),
                 jax.ShapeDtypeStruct((NE, 128), jnp.float32)]
    return pl.pallas_call(body, grid=(NE // BEB,), in_specs=in_specs,
                          out_specs=out_specs, out_shape=out_shape)(*args)


def _combine(pa, pb, skip_x, skip_w, skip_b, lnw, lnb, b8, wmn, wan,
             x1=None, slw=None, slb=None):
    has_sw = skip_w is not None
    has_l = x1 is not None
    skip_d = skip_x.shape[1]

    def body(*refs):
        i = 0
        pa_r = refs[i]; i += 1
        pb_r = refs[i]; i += 1
        sk_r = refs[i]; i += 1
        if has_sw:
            skw = refs[i]; i += 1
            skb = refs[i]; i += 1
        if has_l:
            x1_r = refs[i]; i += 1
            slw_r = refs[i]; i += 1
            slb_r = refs[i]; i += 1
        lnw_r = refs[i]; i += 1
        lnb_r = refs[i]; i += 1
        b8_r = refs[i]; i += 1
        wmn_r = refs[i]; i += 1
        wan_r = refs[i]; i += 1
        x_o, tm_o, ta_o = refs[i], refs[i + 1], refs[i + 2]

        Sb = pb_r[0] + pb_r[1]
        den = _mm(Sb[:, 32:40], b8_r[...]) + 1e-16
        agg = (pa_r[0] + pa_r[1]) / den
        if has_sw:
            sk = _mm(sk_r[...], skw[...]) + skb[...]
        else:
            sk = sk_r[...]
        x = jnp.maximum(_ln(agg + sk, lnw_r[...], lnb_r[...]), 0.0)
        if has_l:
            x = x + _mm(x1_r[...], slw_r[...]) + slb_r[...]
        x_o[...] = x
        tm_o[...] = _mm(x, wmn_r[...])
        ta_o[...] = _mm(x, wan_r[...])

    in_specs = [pl.BlockSpec((2, BN, 128), lambda i: (0, i, 0)),
                pl.BlockSpec((2, BN, 128), lambda i: (0, i, 0)),
                _row(BN, skip_d)]
    args = [pa, pb, skip_x]
    if has_sw:
        in_specs += [_full((skip_d, 128)), _full((1, 128))]
        args += [skip_w, skip_b]
    if has_l:
        in_specs += [_row(BN, 128), _full((128, 128)), _full((1, 128))]
        args += [x1, slw, slb]
    in_specs += [_full((1, 128)), _full((1, 128)), _full((8, 128)),
                 _full((wmn.shape[0], 128)), _full((wan.shape[0], 128))]
    args += [lnw, lnb, b8, wmn, wan]
    return pl.pallas_call(
        body, grid=(NN // BN,), in_specs=in_specs,
        out_specs=[_row(BN, 128), _row(BN, 128), _row(BN, 128)],
        out_shape=[jax.ShapeDtypeStruct((NN, 128), jnp.float32),
                   jax.ShapeDtypeStruct((NN, 128), jnp.float32),
                   jax.ShapeDtypeStruct((NN, 128), jnp.float32)],
    )(*args)


def _edge_head(gso, gdo, e2, ee, b1, w2, b2, w3, b3, w4p, b4p):
    def body(gs_r, gd_r, e_r, eer, b1r, w2r, b2r, w3r, b3r, w4r, b4r, l_o, p_o):
        g = jnp.maximum(gs_r[...] + gd_r[...]
                        + _mm(e_r[...][:, 0:32], eer[...]) + b1r[...], 0.0)
        g = jnp.maximum(_mm(g, w2r[...]) + b2r[...], 0.0)
        g = jnp.maximum(_mm(g, w3r[...]) + b3r[...], 0.0)
        l8 = _mm(g, w4r[...]) + b4r[...]
        l_o[...] = l8
        p_o[...] = 1.0 / (1.0 + jnp.exp(-l8))

    return pl.pallas_call(
        body, grid=(NE // BEB,),
        in_specs=[_row(BEB, 128), _row(BEB, 128), _row(BEB, 128),
                  _full((32, 128)), _full((1, 128)), _full((128, 64)), _full((1, 64)),
                  _full((64, 32)), _full((1, 32)), _full((32, 8)), _full((1, 8))],
        out_specs=[_row(BEB, 8), _row(BEB, 8)],
        out_shape=[jax.ShapeDtypeStruct((NE, 8), jnp.float32),
                   jax.ShapeDtypeStruct((NE, 8), jnp.float32)],
    )(gso, gdo, e2, ee, b1, w2, b2, w3, b3, w4p, b4p)


def _node_head(xo, w1, b1, w2, b2, w3, b3, w4p, b4p):
    def body(x_r, w1r, b1r, w2r, b2r, w3r, b3r, w4r, b4r, l_o, p_o):
        h = jnp.maximum(_mm(x_r[...], w1r[...]) + b1r[...], 0.0)
        h = jnp.maximum(_mm(h, w2r[...]) + b2r[...], 0.0)
        h = jnp.maximum(_mm(h, w3r[...]) + b3r[...], 0.0)
        l8 = _mm(h, w4r[...]) + b4r[...]
        l_o[...] = l8
        z = jnp.exp(l8 - jnp.max(l8, -1, keepdims=True))
        p_o[...] = z / jnp.sum(z, -1, keepdims=True)

    return pl.pallas_call(
        body, grid=(NN // BN,),
        in_specs=[_row(BN, 128),
                  _full((128, 64)), _full((1, 64)), _full((64, 32)), _full((1, 32)),
                  _full((32, 16)), _full((1, 16)), _full((16, 8)), _full((1, 8))],
        out_specs=[_row(BN, 8), _row(BN, 8)],
        out_shape=[jax.ShapeDtypeStruct((NN, 8), jnp.float32),
                   jax.ShapeDtypeStruct((NN, 8), jnp.float32)],
    )(xo, w1, b1, w2, b2, w3, b3, w4p, b4p)


def _r2(b):
    return b.reshape(1, -1)


def _conv_w(cp, nd):
    a1w = cp['a1'][0]
    ai, aj, ae = a1w[:nd], a1w[nd:2 * nd], a1w[2 * nd:]
    m1w = cp['m1'][0]
    mx, me = m1w[:nd], m1w[nd:]
    a2p = jnp.pad(cp['a2'][0], ((0, 0), (0, 8 - HEADS)))
    ba2 = jnp.pad(cp['a2'][1], (0, 8 - HEADS))
    b8 = jnp.repeat(jnp.eye(HEADS, dtype=jnp.float32), HID, axis=1)
    b8 = jnp.pad(b8, ((0, 8 - HEADS), (0, 0)))
    W = {'ae': ae, 'ba1': _r2(cp['a1'][1]), 'a2p': a2p, 'ba2': _r2(ba2),
         'me': me, 'bm1': _r2(cp['m1'][1]), 'm2': cp['m2'][0],
         'bm2': _r2(cp['m2'][1]), 'b8': b8}
    return W, ai, aj, mx


def kernel(x_in, edge_index, edge_attr, params):
    P = params
    src = edge_index[0]
    dst = edge_index[1]

    W1, ai1, aj1, mx1 = _conv_w(P['conv1'], HID)
    W2, ai2, aj2, mx2 = _conv_w(P['conv2'], OUT)
    W3, ai3, aj3, mx3 = _conv_w(P['conv3'], OUT)
    u1w = P['eup1']['w1'][0]
    W2.update({'ue': u1w[2 * OUT:], 'bu1': _r2(P['eup1']['w1'][1]),
               'uw2': P['eup1']['w2'][0], 'bu2': _r2(P['eup1']['w2'][1]),
               'lnw': _r2(P['eup1']['ln'][0]), 'lnb': _r2(P['eup1']['ln'][1])})
    u2w = P['eup2']['w1'][0]
    W3.update({'ue': u2w[2 * OUT:], 'bu1': _r2(P['eup2']['w1'][1]),
               'uw2': P['eup2']['w2'][0], 'bu2': _r2(P['eup2']['w2'][1]),
               'lnw': _r2(P['eup2']['ln'][0]), 'lnb': _r2(P['eup2']['ln'][1])})

    wm0 = mx1
    wa0 = jnp.concatenate([aj1, ai1, jnp.zeros((HID, 64), jnp.float32)], 1)
    wa1 = jnp.concatenate([aj2, u1w[OUT:2 * OUT], ai2, u1w[:OUT]], 1)
    wa2 = jnp.concatenate([aj3, u2w[OUT:2 * OUT], ai3, u2w[:OUT]], 1)
    ehw = P['eh1'][0]
    wso, wdo = ehw[:OUT], ehw[OUT:2 * OUT]

    x0, tm0, ta0 = _node_encode(
        x_in, _r2(P['dummy']), P['ne1'][0], _r2(P['ne1'][1]),
        P['ne2'][0], _r2(P['ne2'][1]), wm0, wa0)
    e0 = _edge_encode(edge_attr, P['ee1'][0], _r2(P['ee1'][1]),
                      P['ee2'][0], _r2(P['ee2'][1]))

    gm, gas, gad = _sc_gather_multi([tm0, ta0, ta0], [0, 0, 1], src, dst)
    wexa, comb1 = _edge_stage(gm, gas, gad, e0, W1, False)
    pa = _sc_scatter(wexa, dst, 128, True)
    pb = _sc_scatter(comb1, dst, 128, True)
    x1, tm1, ta1 = _combine(pa, pb, x0, P['skip0'][0], _r2(P['skip0'][1]),
                            _r2(P['ln1'][0]), _r2(P['ln1'][1]), W1['b8'], mx2, wa1)

    gm, gas, gad = _sc_gather_multi([tm1, ta1, ta1], [0, 0, 1], src, dst)
    wexa, comb2 = _edge_stage(gm, gas, gad, e0, W2, True)
    pa = _sc_scatter(wexa, dst, 128, True)
    pb = _sc_scatter(comb2, dst, 128, True)
    x2, tm2, ta2 = _combine(pa, pb, x1, None, None,
                            _r2(P['ln2'][0]), _r2(P['ln2'][1]), W1['b8'], mx3, wa2)

    gm, gas, gad = _sc_gather_multi([tm2, ta2, ta2], [0, 0, 1], src, dst)
    wexa, comb3 = _edge_stage(gm, gas, gad, comb2, W3, True)
    pa = _sc_scatter(wexa, dst, 128, True)
    pb = _sc_scatter(comb3, dst, 128, True)
    xo, tso, tdo = _combine(pa, pb, x2, None, None,
                            _r2(P['ln3'][0]), _r2(P['ln3'][1]), W1['b8'], wso, wdo,
                            x1=x1, slw=P['skipL'][0], slb=_r2(P['skipL'][1]))

    gso, gdo = _sc_gather_multi([tso, tdo], [0, 1], src, dst)
    el8, ep8 = _edge_head(gso, gdo, comb3, ehw[2 * OUT:], _r2(P['eh1'][1]),
                          P['eh2'][0], _r2(P['eh2'][1]),
                          P['eh3'][0], _r2(P['eh3'][1]),
                          jnp.pad(P['eh4'][0], ((0, 0), (0, 7))),
                          _r2(jnp.pad(P['eh4'][1], (0, 7))))

    nb4 = jnp.pad(P['nh4'][1], (0, 1), constant_values=-1e30)
    nl8, np8 = _node_head(xo, P['nh1'][0], _r2(P['nh1'][1]),
                          P['nh2'][0], _r2(P['nh2'][1]),
                          P['nh3'][0], _r2(P['nh3'][1]),
                          jnp.pad(P['nh4'][0], ((0, 0), (0, 1))), _r2(nb4))

    return (nl8[:, :7], el8[:, :1], np8[:, :7], ep8[:, :1])
